# baseline (speedup 1.0000x reference)
import functools

import jax
import jax.numpy as jnp
from jax import lax
from jax.experimental import pallas as pl
from jax.experimental.pallas import tpu as pltpu

_B = 8
_A = 5
_NATT = 85
_HW = 1024
_N = _A * _HW
_K = 1000
_KPAD = 1024
_STRIDE = 32.0
_NMS_THR = 0.5

_JC = 128
_IC = 128


def _sig(x):
    return jax.nn.sigmoid(x)


def _yolof_body(pred_ref, anch_ref, out_ref, d_ref, sl_ref, ss_ref):
    for a in range(_A):
        pa = pred_ref[0, a * _NATT:(a + 1) * _NATT, :]
        an = anch_ref[a * 4:(a + 1) * 4, :]
        cx = _sig(pa[0:1, :]) * _STRIDE + an[0:1, :]
        cy = _sig(pa[1:2, :]) * _STRIDE + an[1:2, :]
        w = jnp.exp(jnp.clip(pa[2:3, :], -10.0, 8.0)) * an[2:3, :]
        h = jnp.exp(jnp.clip(pa[3:4, :], -10.0, 8.0)) * an[3:4, :]
        cls_max = jnp.max(pa[5:_NATT, :], axis=0, keepdims=True)
        score = _sig(pa[4:5, :]) * _sig(cls_max)
        sl = pl.ds(a * _HW, _HW)
        d_ref[0:1, sl] = cx - w * 0.5
        d_ref[1:2, sl] = cy - h * 0.5
        d_ref[2:3, sl] = cx + w * 0.5
        d_ref[3:4, sl] = cy + h * 0.5
        d_ref[4:5, sl] = score
    d_ref[5:8, :] = jnp.zeros((3, _N), jnp.float32)

    s_row = d_ref[4:5, :]
    ii = lax.broadcasted_iota(jnp.int32, (1, _N), 1)

    def rank_step(jc, rank):
        sj = jnp.transpose(d_ref[4:5, pl.ds(jc * _JC, _JC)])
        jj = jc * _JC + lax.broadcasted_iota(jnp.int32, (_JC, 1), 0)
        gt = sj > s_row
        tie = (sj == s_row) & (jj < ii)
        cnt = jnp.sum((gt | tie).astype(jnp.int32), axis=0, keepdims=True)
        return rank + cnt

    rank = lax.fori_loop(0, _N // _JC, rank_step,
                         jnp.zeros((1, _N), jnp.int32))

    def gather_step(kc, _):
        kk = kc * _JC + lax.broadcasted_iota(jnp.int32, (_JC, 1), 0)
        onehot = (rank == kk).astype(jnp.float32)
        d_all = d_ref[...]
        lane = lax.dot_general(d_all, onehot, (((1,), (1,)), ((), ())),
                               preferred_element_type=jnp.float32)
        sub = lax.dot_general(onehot, d_all, (((1,), (1,)), ((), ())),
                              preferred_element_type=jnp.float32)
        sl_ref[:, pl.ds(kc * _JC, _JC)] = lane
        ss_ref[pl.ds(kc * _JC, _JC), :] = sub
        return 0

    lax.fori_loop(0, _KPAD // _JC, gather_step, 0)

    xj1 = sl_ref[0:1, :]
    yj1 = sl_ref[1:2, :]
    xj2 = sl_ref[2:3, :]
    yj2 = sl_ref[3:4, :]
    aj = (xj2 - xj1) * (yj2 - yj1)
    colid = lax.broadcasted_iota(jnp.int32, (1, _KPAD), 1)

    def nms_step(ic, colmax):
        rs = pl.ds(ic * _IC, _IC)
        xi1 = ss_ref[rs, 0:1]
        yi1 = ss_ref[rs, 1:2]
        xi2 = ss_ref[rs, 2:3]
        yi2 = ss_ref[rs, 3:4]
        ix = jnp.clip(jnp.minimum(xi2, xj2) - jnp.maximum(xi1, xj1), 0.0, None)
        iy = jnp.clip(jnp.minimum(yi2, yj2) - jnp.maximum(yi1, yj1), 0.0, None)
        inter = ix * iy
        ai = (xi2 - xi1) * (yi2 - yi1)
        iou = inter / jnp.maximum(ai + aj - inter, 1e-6)
        rowid = ic * _IC + lax.broadcasted_iota(jnp.int32, (_IC, 1), 0)
        iou = jnp.where(rowid < colid, iou, 0.0)
        return jnp.maximum(colmax, jnp.max(iou, axis=0, keepdims=True))

    colmax = lax.fori_loop(0, _KPAD // _IC, nms_step,
                           jnp.zeros((1, _KPAD), jnp.float32))
    keep = (colmax <= _NMS_THR).astype(jnp.float32)

    out_ref[0, 0:4, :] = sl_ref[0:4, :]
    out_ref[0, 4:5, :] = sl_ref[4:5, :] * keep
    out_ref[0, 5:8, :] = jnp.zeros((3, _KPAD), jnp.float32)


@jax.jit
def kernel(pred_map, anchors):
    pm = pred_map.reshape(_B, _A * _NATT, _HW)
    an = anchors.reshape(_A, _HW, 4).transpose(0, 2, 1).reshape(_A * 4, _HW)
    out = pl.pallas_call(
        _yolof_body,
        grid=(_B,),
        in_specs=[
            pl.BlockSpec((1, _A * _NATT, _HW), lambda b: (b, 0, 0)),
            pl.BlockSpec((_A * 4, _HW), lambda b: (0, 0)),
        ],
        out_specs=pl.BlockSpec((1, 8, _KPAD), lambda b: (b, 0, 0)),
        out_shape=jax.ShapeDtypeStruct((_B, 8, _KPAD), jnp.float32),
        scratch_shapes=[
            pltpu.VMEM((8, _N), jnp.float32),
            pltpu.VMEM((8, _KPAD), jnp.float32),
            pltpu.VMEM((_KPAD, 8), jnp.float32),
        ],
    )(pm, an)
    return jnp.transpose(out, (0, 2, 1))[:, :_K, :5]
